# baseline (speedup 1.0000x reference)
import jax
import jax.numpy as jnp
from jax.experimental import pallas as pl

N_P = 8
N_S = 64
N_A = N_P + N_S
COMM_RANGE = 0.3
HID = 256
BG = 8


def _body(obs_ref, ph_ref, s_ref, wq_ref, wk_ref, wv_ref, fcw_ref, fcb_ref,
          h_out_ref, alpha_ref):
    ph = ph_ref[...]
    s = s_ref[...]
    R = ph.shape[0]
    C = s.shape[0]

    ob = obs_ref[...]
    xc = ob[:, 0:1].reshape(BG, N_A, 1)
    yc = ob[:, 1:2].reshape(BG, N_A, 1)
    px = xc[:, :N_P, :].reshape(R, 1)
    py = yc[:, :N_P, :].reshape(R, 1)
    sx = xc[:, N_P:, :].reshape(C, 1).reshape(1, C)
    sy = yc[:, N_P:, :].reshape(C, 1).reshape(1, C)

    q = jnp.dot(ph, wq_ref[...], preferred_element_type=jnp.float32)
    k = jnp.dot(s, wk_ref[...], preferred_element_type=jnp.float32)
    v = jnp.dot(s, wv_ref[...], preferred_element_type=jnp.float32)

    e = jax.lax.dot_general(q, k, (((1,), (1,)), ((), ())),
                            preferred_element_type=jnp.float32)
    e = e * (1.0 / jnp.sqrt(jnp.float32(HID)))

    dx = jnp.abs(px - sx)
    dy = jnp.abs(py - sy)
    near = jnp.maximum(dx, dy) <= COMM_RANGE
    rowb = jax.lax.broadcasted_iota(jnp.int32, (R, C), 0) // N_P
    colb = jax.lax.broadcasted_iota(jnp.int32, (R, C), 1) // N_S
    mask = near & (rowb == colb)

    e = jnp.where(mask, e, -1e30)
    m = jnp.max(e, axis=1, keepdims=True)
    ex = jnp.exp(e - m)
    a = ex / jnp.sum(ex, axis=1, keepdims=True)
    a = jnp.where(mask, a, 0.0)
    has_vis = jnp.any(mask, axis=1, keepdims=True)

    attn = jnp.dot(a, v, preferred_element_type=jnp.float32)
    h = jnp.where(has_vis, attn, ph)
    h_out_ref[...] = jnp.dot(h, fcw_ref[...],
                             preferred_element_type=jnp.float32) + fcb_ref[...]

    sel = (jax.lax.broadcasted_iota(jnp.int32, (C, N_S), 0) % N_S ==
           jax.lax.broadcasted_iota(jnp.int32, (C, N_S), 1))
    alpha_ref[...] = jnp.dot(a, sel.astype(jnp.float32),
                             preferred_element_type=jnp.float32)


def kernel(obs, p_hidden, s_hidden, batch_size, Wq, Wk, Wv, fc_W, fc_b):
    B = p_hidden.shape[0] // N_P
    G = B // BG
    obs2d = obs.reshape(B * N_A, 8)
    s_flat = s_hidden.reshape(B * N_S, HID)
    fc_b2 = fc_b.reshape(1, HID)

    const2d = pl.BlockSpec((HID, HID), lambda g: (0, 0))
    h_out, alpha = pl.pallas_call(
        _body,
        grid=(G,),
        in_specs=[
            pl.BlockSpec((BG * N_A, 8), lambda g: (g, 0)),
            pl.BlockSpec((BG * N_P, HID), lambda g: (g, 0)),
            pl.BlockSpec((BG * N_S, HID), lambda g: (g, 0)),
            const2d, const2d, const2d, const2d,
            pl.BlockSpec((1, HID), lambda g: (0, 0)),
        ],
        out_specs=[
            pl.BlockSpec((BG * N_P, HID), lambda g: (g, 0)),
            pl.BlockSpec((BG * N_P, N_S), lambda g: (g, 0)),
        ],
        out_shape=[
            jax.ShapeDtypeStruct((B * N_P, HID), jnp.float32),
            jax.ShapeDtypeStruct((B * N_P, N_S), jnp.float32),
        ],
    )(obs2d, p_hidden, s_flat, Wq, Wk, Wv, fc_W, fc_b2)
    return h_out.reshape(B, N_P, HID), alpha.reshape(B, N_P, N_S)


# speedup vs baseline: 1.1479x; 1.1479x over previous
import jax
import jax.numpy as jnp
from jax.experimental import pallas as pl

N_P = 8
N_S = 64
N_A = N_P + N_S
COMM_RANGE = 0.3
HID = 256
BG = 16


def _body(obs_ref, ph_ref, s_ref, wq_ref, wk_ref, wv_ref, fcw_ref, fcb_ref,
          h_out_ref, alpha_ref):
    ph = ph_ref[...]
    s = s_ref[...]
    R = ph.shape[0]
    C = s.shape[0]

    ob = obs_ref[...]
    xc = ob[:, 0:1].reshape(BG, N_A, 1)
    yc = ob[:, 1:2].reshape(BG, N_A, 1)
    px = xc[:, :N_P, :].reshape(R, 1)
    py = yc[:, :N_P, :].reshape(R, 1)
    sx = xc[:, N_P:, :].reshape(C, 1).reshape(1, C)
    sy = yc[:, N_P:, :].reshape(C, 1).reshape(1, C)

    q = jnp.dot(ph, wq_ref[...], preferred_element_type=jnp.float32)
    k = jnp.dot(s, wk_ref[...], preferred_element_type=jnp.float32)
    v = jnp.dot(s, wv_ref[...], preferred_element_type=jnp.float32)

    e = jax.lax.dot_general(q, k, (((1,), (1,)), ((), ())),
                            preferred_element_type=jnp.float32)
    e = e * (1.0 / jnp.sqrt(jnp.float32(HID)))

    dx = jnp.abs(px - sx)
    dy = jnp.abs(py - sy)
    near = jnp.maximum(dx, dy) <= COMM_RANGE
    rowb = jax.lax.broadcasted_iota(jnp.int32, (R, C), 0) // N_P
    colb = jax.lax.broadcasted_iota(jnp.int32, (R, C), 1) // N_S
    mask = near & (rowb == colb)

    e = jnp.where(mask, e, -1e30)
    m = jnp.max(e, axis=1, keepdims=True)
    ex = jnp.exp(e - m)
    a = ex / jnp.sum(ex, axis=1, keepdims=True)
    a = jnp.where(mask, a, 0.0)
    has_vis = jnp.any(mask, axis=1, keepdims=True)

    attn = jnp.dot(a, v, preferred_element_type=jnp.float32)
    h = jnp.where(has_vis, attn, ph)
    h_out_ref[...] = jnp.dot(h, fcw_ref[...],
                             preferred_element_type=jnp.float32) + fcb_ref[...]

    sel = (jax.lax.broadcasted_iota(jnp.int32, (C, N_S), 0) % N_S ==
           jax.lax.broadcasted_iota(jnp.int32, (C, N_S), 1))
    alpha_ref[...] = jnp.dot(a, sel.astype(jnp.float32),
                             preferred_element_type=jnp.float32)


def kernel(obs, p_hidden, s_hidden, batch_size, Wq, Wk, Wv, fc_W, fc_b):
    B = p_hidden.shape[0] // N_P
    G = B // BG
    obs2d = obs.reshape(B * N_A, 8)
    s_flat = s_hidden.reshape(B * N_S, HID)
    fc_b2 = fc_b.reshape(1, HID)

    const2d = pl.BlockSpec((HID, HID), lambda g: (0, 0))
    h_out, alpha = pl.pallas_call(
        _body,
        grid=(G,),
        in_specs=[
            pl.BlockSpec((BG * N_A, 8), lambda g: (g, 0)),
            pl.BlockSpec((BG * N_P, HID), lambda g: (g, 0)),
            pl.BlockSpec((BG * N_S, HID), lambda g: (g, 0)),
            const2d, const2d, const2d, const2d,
            pl.BlockSpec((1, HID), lambda g: (0, 0)),
        ],
        out_specs=[
            pl.BlockSpec((BG * N_P, HID), lambda g: (g, 0)),
            pl.BlockSpec((BG * N_P, N_S), lambda g: (g, 0)),
        ],
        out_shape=[
            jax.ShapeDtypeStruct((B * N_P, HID), jnp.float32),
            jax.ShapeDtypeStruct((B * N_P, N_S), jnp.float32),
        ],
    )(obs2d, p_hidden, s_flat, Wq, Wk, Wv, fc_W, fc_b2)
    return h_out.reshape(B, N_P, HID), alpha.reshape(B, N_P, N_S)


# speedup vs baseline: 1.1497x; 1.0015x over previous
import jax
import jax.numpy as jnp
from jax.experimental import pallas as pl

N_P = 8
N_S = 64
N_A = N_P + N_S
COMM_RANGE = 0.3
HID = 256
BG = 16


def _body(obs_ref, ph_ref, s_ref, wq_ref, wk_ref, wv_ref, fcw_ref, fcb_ref,
          h_out_ref, alpha_ref):
    ph = ph_ref[...]
    s = s_ref[...]
    R = ph.shape[0]
    C = s.shape[0]

    ob = obs_ref[...]
    xc = ob[:, 0:1].reshape(BG, N_A, 1)
    yc = ob[:, 1:2].reshape(BG, N_A, 1)
    px = xc[:, :N_P, :].reshape(R, 1)
    py = yc[:, :N_P, :].reshape(R, 1)
    sx = xc[:, N_P:, :].reshape(C, 1).reshape(1, C)
    sy = yc[:, N_P:, :].reshape(C, 1).reshape(1, C)

    sb = s.astype(jnp.bfloat16)
    q = jnp.dot(ph.astype(jnp.bfloat16), wq_ref[...].astype(jnp.bfloat16),
                preferred_element_type=jnp.float32)
    k = jnp.dot(sb, wk_ref[...].astype(jnp.bfloat16),
                preferred_element_type=jnp.float32)
    v = jnp.dot(sb, wv_ref[...].astype(jnp.bfloat16),
                preferred_element_type=jnp.float32)

    e = jax.lax.dot_general(q.astype(jnp.bfloat16), k.astype(jnp.bfloat16),
                            (((1,), (1,)), ((), ())),
                            preferred_element_type=jnp.float32)
    e = e * (1.0 / jnp.sqrt(jnp.float32(HID)))

    dx = jnp.abs(px - sx)
    dy = jnp.abs(py - sy)
    near = jnp.maximum(dx, dy) <= COMM_RANGE
    rowb = jax.lax.broadcasted_iota(jnp.int32, (R, C), 0) // N_P
    colb = jax.lax.broadcasted_iota(jnp.int32, (R, C), 1) // N_S
    mask = near & (rowb == colb)

    e = jnp.where(mask, e, -1e30)
    m = jnp.max(e, axis=1, keepdims=True)
    ex = jnp.exp(e - m)
    a = ex / jnp.sum(ex, axis=1, keepdims=True)
    a = jnp.where(mask, a, 0.0)
    has_vis = jnp.any(mask, axis=1, keepdims=True)

    attn = jnp.dot(a.astype(jnp.bfloat16), v.astype(jnp.bfloat16),
                   preferred_element_type=jnp.float32)
    h = jnp.where(has_vis, attn, ph)
    h_out_ref[...] = jnp.dot(h.astype(jnp.bfloat16),
                             fcw_ref[...].astype(jnp.bfloat16),
                             preferred_element_type=jnp.float32) + fcb_ref[...]

    sel = (jax.lax.broadcasted_iota(jnp.int32, (C, N_S), 0) % N_S ==
           jax.lax.broadcasted_iota(jnp.int32, (C, N_S), 1))
    alpha_ref[...] = jnp.dot(a, sel.astype(jnp.float32),
                             preferred_element_type=jnp.float32)


def kernel(obs, p_hidden, s_hidden, batch_size, Wq, Wk, Wv, fc_W, fc_b):
    B = p_hidden.shape[0] // N_P
    G = B // BG
    obs2d = obs.reshape(B * N_A, 8)
    s_flat = s_hidden.reshape(B * N_S, HID)
    fc_b2 = fc_b.reshape(1, HID)

    const2d = pl.BlockSpec((HID, HID), lambda g: (0, 0))
    h_out, alpha = pl.pallas_call(
        _body,
        grid=(G,),
        in_specs=[
            pl.BlockSpec((BG * N_A, 8), lambda g: (g, 0)),
            pl.BlockSpec((BG * N_P, HID), lambda g: (g, 0)),
            pl.BlockSpec((BG * N_S, HID), lambda g: (g, 0)),
            const2d, const2d, const2d, const2d,
            pl.BlockSpec((1, HID), lambda g: (0, 0)),
        ],
        out_specs=[
            pl.BlockSpec((BG * N_P, HID), lambda g: (g, 0)),
            pl.BlockSpec((BG * N_P, N_S), lambda g: (g, 0)),
        ],
        out_shape=[
            jax.ShapeDtypeStruct((B * N_P, HID), jnp.float32),
            jax.ShapeDtypeStruct((B * N_P, N_S), jnp.float32),
        ],
    )(obs2d, p_hidden, s_flat, Wq, Wk, Wv, fc_W, fc_b2)
    return h_out.reshape(B, N_P, HID), alpha.reshape(B, N_P, N_S)
